# Initial kernel scaffold; baseline (speedup 1.0000x reference)
import jax
import jax.numpy as jnp
from jax.experimental import pallas as pl


def kernel(x, edge_index, W1, b1, W2, b2, Wf, bf):
    raise NotImplementedError("write your pallas kernel here")


# speedup vs baseline: 22.3521x; 22.3521x over previous
import jax
import jax.numpy as jnp
from jax import lax
from jax.experimental import pallas as pl
from jax.experimental.pallas import tpu as pltpu
from jax.experimental.pallas import tpu_sc as plsc

N = 10000
E = 320000
D_IN = 128
D_HID = 64
D_OUT = 4

NC = 2
NS = 16
NW = NC * NS
EPW = E // NW
CHUNK = 80
NCHUNK = EPW // CHUNK
NPAD = 10240
SEG = NPAD // NS
ROWS_PT = NPAD // NS
OUTP = 128


def _mesh():
    return plsc.VectorSubcoreMesh(core_axis_name="c", subcore_axis_name="s")


def _deg_body(dst_hbm, zeros_hbm, deg_out, hist_v, idx_v, seg_v, out_v,
              shared_sm):
    c = lax.axis_index("c")
    s = lax.axis_index("s")
    wid = c * NS + s
    pltpu.sync_copy(zeros_hbm, hist_v)
    pltpu.sync_copy(dst_hbm.at[wid], idx_v)
    ones = jnp.full((16,), 1.0, jnp.float32)

    def body(i, carry):
        idx = idx_v[pl.ds(i * 16, 16)]
        plsc.addupdate_scatter(hist_v, [idx], ones)
        return carry

    lax.fori_loop(0, EPW // 16, body, 0)

    for t in range(NS):
        pltpu.sync_copy(hist_v.at[pl.ds(t * SEG, SEG)], shared_sm.at[s, t])
    plsc.subcore_barrier()
    for t in range(NS):
        pltpu.sync_copy(shared_sm.at[t, s], seg_v.at[t])

    def red(k, carry):
        v = jnp.zeros((16,), jnp.float32)
        for t in range(NS):
            v = v + seg_v[t, pl.ds(k * 16, 16)]
        out_v[pl.ds(k * 16, 16)] = v
        return carry

    lax.fori_loop(0, SEG // 16, red, 0)
    pltpu.sync_copy(out_v, deg_out.at[c, pl.ds(s * SEG, SEG)])


_deg = pl.kernel(
    _deg_body,
    out_type=jax.ShapeDtypeStruct((NC, NPAD), jnp.float32),
    mesh=_mesh(),
    scratch_types=[
        pltpu.VMEM((NPAD,), jnp.float32),
        pltpu.VMEM((EPW,), jnp.int32),
        pltpu.VMEM((NS, SEG), jnp.float32),
        pltpu.VMEM((SEG,), jnp.float32),
        pltpu.MemorySpace.VMEM_SHARED((NS, NS, SEG), jnp.float32),
    ],
    compiler_params=pltpu.CompilerParams(needs_layout_passes=False),
)


def _mp_body(hs_hbm, src_hbm, dst_hbm, zeros_hbm, out_hbm, isrc_v, idst_v,
             rows_v, acc_sm, sem):
    c = lax.axis_index("c")
    s = lax.axis_index("s")
    wid = c * NS + s
    r0 = s * ROWS_PT
    pltpu.sync_copy(zeros_hbm.at[pl.ds(r0, ROWS_PT)],
                    acc_sm.at[pl.ds(r0, ROWS_PT)])
    pltpu.sync_copy(src_hbm.at[wid], isrc_v)
    pltpu.sync_copy(dst_hbm.at[wid], idst_v)
    plsc.subcore_barrier()

    def body(j, carry):
        pltpu.async_copy(hs_hbm.at[isrc_v.at[j]], rows_v, sem).wait()
        pltpu.sync_copy(rows_v, acc_sm.at[idst_v.at[j]], add=True)
        return carry

    lax.fori_loop(0, NCHUNK, body, 0)
    plsc.subcore_barrier()
    pltpu.sync_copy(acc_sm.at[pl.ds(r0, ROWS_PT)],
                    out_hbm.at[c, pl.ds(r0, ROWS_PT)])


_mp = pl.kernel(
    _mp_body,
    out_type=jax.ShapeDtypeStruct((NC, NPAD, D_HID), jnp.float32),
    mesh=_mesh(),
    scratch_types=[
        pltpu.VMEM((NCHUNK, CHUNK), jnp.int32),
        pltpu.VMEM((NCHUNK, CHUNK), jnp.int32),
        pltpu.VMEM((CHUNK, D_HID), jnp.float32),
        pltpu.MemorySpace.VMEM_SHARED((NPAD, D_HID), jnp.float32),
        pltpu.SemaphoreType.DMA,
    ],
    compiler_params=pltpu.CompilerParams(needs_layout_passes=False,
                                         use_tc_tiling_on_sc=False),
)


BLK = 640


def _tc1_body(x_ref, w_ref, d0_ref, d1_ref, hs_ref, dinv_ref):
    deg = d0_ref[...] + d1_ref[...] + 1.0
    dinv = lax.rsqrt(deg)
    h = jnp.dot(x_ref[...], w_ref[...], preferred_element_type=jnp.float32)
    hs_ref[...] = h * dinv
    dinv_ref[...] = dinv


_tc1 = pl.pallas_call(
    _tc1_body,
    grid=(NPAD // BLK,),
    in_specs=[
        pl.BlockSpec((BLK, D_IN), lambda i: (i, 0)),
        pl.BlockSpec((D_IN, D_HID), lambda i: (0, 0)),
        pl.BlockSpec((BLK, 1), lambda i: (i, 0)),
        pl.BlockSpec((BLK, 1), lambda i: (i, 0)),
    ],
    out_specs=[
        pl.BlockSpec((BLK, D_HID), lambda i: (i, 0)),
        pl.BlockSpec((BLK, 1), lambda i: (i, 0)),
    ],
    out_shape=[
        jax.ShapeDtypeStruct((NPAD, D_HID), jnp.float32),
        jax.ShapeDtypeStruct((NPAD, 1), jnp.float32),
    ],
)


def _tc2_body(a0_ref, a1_ref, hs_ref, dinv_ref, b_ref, w_ref, out_ref):
    dinv = dinv_ref[...]
    pre = (a0_ref[...] + a1_ref[...] + hs_ref[...]) * dinv + b_ref[...]
    t = jnp.maximum(pre, 0.0)
    out_ref[...] = jnp.dot(t, w_ref[...],
                           preferred_element_type=jnp.float32) * dinv


_tc2 = pl.pallas_call(
    _tc2_body,
    grid=(NPAD // BLK,),
    in_specs=[
        pl.BlockSpec((BLK, D_HID), lambda i: (i, 0)),
        pl.BlockSpec((BLK, D_HID), lambda i: (i, 0)),
        pl.BlockSpec((BLK, D_HID), lambda i: (i, 0)),
        pl.BlockSpec((BLK, 1), lambda i: (i, 0)),
        pl.BlockSpec((1, D_HID), lambda i: (0, 0)),
        pl.BlockSpec((D_HID, D_HID), lambda i: (0, 0)),
    ],
    out_specs=pl.BlockSpec((BLK, D_HID), lambda i: (i, 0)),
    out_shape=jax.ShapeDtypeStruct((NPAD, D_HID), jnp.float32),
)


def _tc3_body(a0_ref, a1_ref, hs_ref, dinv_ref, b_ref, w_ref, bf_ref,
              out_ref):
    dinv = dinv_ref[...]
    pre = (a0_ref[...] + a1_ref[...] + hs_ref[...]) * dinv + b_ref[...]
    t = jnp.maximum(pre, 0.0)
    out_ref[...] = jnp.dot(t, w_ref[...],
                           preferred_element_type=jnp.float32) + bf_ref[...]


_tc3 = pl.pallas_call(
    _tc3_body,
    grid=(NPAD // BLK,),
    in_specs=[
        pl.BlockSpec((BLK, D_HID), lambda i: (i, 0)),
        pl.BlockSpec((BLK, D_HID), lambda i: (i, 0)),
        pl.BlockSpec((BLK, D_HID), lambda i: (i, 0)),
        pl.BlockSpec((BLK, 1), lambda i: (i, 0)),
        pl.BlockSpec((1, D_HID), lambda i: (0, 0)),
        pl.BlockSpec((D_HID, OUTP), lambda i: (0, 0)),
        pl.BlockSpec((1, OUTP), lambda i: (0, 0)),
    ],
    out_specs=pl.BlockSpec((BLK, OUTP), lambda i: (i, 0)),
    out_shape=jax.ShapeDtypeStruct((NPAD, OUTP), jnp.float32),
)


@jax.jit
def kernel(x, edge_index, W1, b1, W2, b2, Wf, bf):
    src = edge_index[0].astype(jnp.int32).reshape(NW, NCHUNK, CHUNK)
    dst = edge_index[1].astype(jnp.int32).reshape(NW, NCHUNK, CHUNK)
    dst_flat = dst.reshape(NW, EPW)
    zpad = jnp.zeros((NPAD,), jnp.float32)
    znd = jnp.zeros((NPAD, D_HID), jnp.float32)
    xp = jnp.pad(x, ((0, NPAD - N), (0, 0)))

    deg_parts = _deg(dst_flat, zpad)
    d0 = deg_parts[0].reshape(NPAD, 1)
    d1 = deg_parts[1].reshape(NPAD, 1)

    hs1, dinv = _tc1(xp, W1, d0, d1)
    acc1 = _mp(hs1, src, dst, znd)
    hs2 = _tc2(acc1[0], acc1[1], hs1, dinv, b1.reshape(1, D_HID), W2)
    acc2 = _mp(hs2, src, dst, znd)

    Wfp = jnp.zeros((D_HID, OUTP), jnp.float32).at[:, :D_OUT].set(Wf)
    bfp = jnp.zeros((1, OUTP), jnp.float32).at[:, :D_OUT].set(bf)
    outp = _tc3(acc2[0], acc2[1], hs2, dinv, b2.reshape(1, D_HID), Wfp, bfp)
    return outp[:N, :D_OUT]


# speedup vs baseline: 36.0241x; 1.6117x over previous
import jax
import jax.numpy as jnp
from jax import lax
from jax.experimental import pallas as pl
from jax.experimental.pallas import tpu as pltpu
from jax.experimental.pallas import tpu_sc as plsc

N = 10000
E = 320000
D_IN = 128
D_HID = 64
D_OUT = 4

NC = 2
NS = 16
NW = NC * NS
EPW = E // NW
CHUNK = 80
NCHUNK = EPW // CHUNK
NPAD = 10240
SEG = NPAD // NS
ROWS_PT = NPAD // NS
OUTP = 128


def _mesh():
    return plsc.VectorSubcoreMesh(core_axis_name="c", subcore_axis_name="s")


def _deg_body(dst_hbm, zeros_hbm, deg_out, hist_v, idx_v, seg_v, out_v,
              shared_sm):
    c = lax.axis_index("c")
    s = lax.axis_index("s")
    wid = c * NS + s
    pltpu.sync_copy(zeros_hbm, hist_v)
    pltpu.sync_copy(dst_hbm.at[wid], idx_v)
    ones = jnp.full((16,), 1.0, jnp.float32)

    def body(i, carry):
        idx = idx_v[pl.ds(i * 16, 16)]
        plsc.addupdate_scatter(hist_v, [idx], ones)
        return carry

    lax.fori_loop(0, EPW // 16, body, 0)

    for t in range(NS):
        pltpu.sync_copy(hist_v.at[pl.ds(t * SEG, SEG)], shared_sm.at[s, t])
    plsc.subcore_barrier()
    for t in range(NS):
        pltpu.sync_copy(shared_sm.at[t, s], seg_v.at[t])

    def red(k, carry):
        v = jnp.zeros((16,), jnp.float32)
        for t in range(NS):
            v = v + seg_v[t, pl.ds(k * 16, 16)]
        out_v[pl.ds(k * 16, 16)] = v
        return carry

    lax.fori_loop(0, SEG // 16, red, 0)
    pltpu.sync_copy(out_v, deg_out.at[c, pl.ds(s * SEG, SEG)])


_deg = pl.kernel(
    _deg_body,
    out_type=jax.ShapeDtypeStruct((NC, NPAD), jnp.float32),
    mesh=_mesh(),
    scratch_types=[
        pltpu.VMEM((NPAD,), jnp.float32),
        pltpu.VMEM((EPW,), jnp.int32),
        pltpu.VMEM((NS, SEG), jnp.float32),
        pltpu.VMEM((SEG,), jnp.float32),
        pltpu.MemorySpace.VMEM_SHARED((NS, NS, SEG), jnp.float32),
    ],
    compiler_params=pltpu.CompilerParams(needs_layout_passes=False),
)


NB = 5
NGROUP = NCHUNK // NB


def _mp_body(hs_hbm, src_hbm, dst_hbm, zeros_hbm, out_hbm, isrc_v, idst_v,
             rows_v, acc_sm, gsem, ssem):
    c = lax.axis_index("c")
    s = lax.axis_index("s")
    wid = c * NS + s
    r0 = s * ROWS_PT
    pltpu.sync_copy(zeros_hbm.at[pl.ds(r0, ROWS_PT)],
                    acc_sm.at[pl.ds(r0, ROWS_PT)])
    pltpu.sync_copy(src_hbm.at[wid], isrc_v)
    pltpu.sync_copy(dst_hbm.at[wid], idst_v)
    plsc.subcore_barrier()

    for b in range(NB):
        pltpu.async_copy(hs_hbm.at[isrc_v.at[b]], rows_v.at[b], gsem.at[b])

    def body(g, carry):
        for b in range(NB):
            j = g * NB + b
            pltpu.make_async_copy(hs_hbm.at[isrc_v.at[j]], rows_v.at[b],
                                  gsem.at[b]).wait()
            pltpu.async_copy(rows_v.at[b], acc_sm.at[idst_v.at[j]],
                             ssem.at[b], add=True)

        @pl.when(g < NGROUP - 1)
        def _():
            for b in range(NB):
                j = g * NB + b
                jn = j + NB
                pltpu.make_async_copy(rows_v.at[b],
                                      acc_sm.at[idst_v.at[j]],
                                      ssem.at[b]).wait()
                pltpu.async_copy(hs_hbm.at[isrc_v.at[jn]], rows_v.at[b],
                                 gsem.at[b])
        return carry

    lax.fori_loop(0, NGROUP, body, 0)
    for b in range(NB):
        j = (NGROUP - 1) * NB + b
        pltpu.make_async_copy(rows_v.at[b], acc_sm.at[idst_v.at[j]],
                              ssem.at[b]).wait()
    plsc.subcore_barrier()
    pltpu.sync_copy(acc_sm.at[pl.ds(r0, ROWS_PT)],
                    out_hbm.at[c, pl.ds(r0, ROWS_PT)])


_mp = pl.kernel(
    _mp_body,
    out_type=jax.ShapeDtypeStruct((NC, NPAD, D_HID), jnp.float32),
    mesh=_mesh(),
    scratch_types=[
        pltpu.VMEM((NCHUNK, CHUNK), jnp.int32),
        pltpu.VMEM((NCHUNK, CHUNK), jnp.int32),
        pltpu.VMEM((NB, CHUNK, D_HID), jnp.float32),
        pltpu.MemorySpace.VMEM_SHARED((NPAD, D_HID), jnp.float32),
        pltpu.SemaphoreType.DMA((NB,)),
        pltpu.SemaphoreType.DMA((NB,)),
    ],
    compiler_params=pltpu.CompilerParams(needs_layout_passes=False,
                                         use_tc_tiling_on_sc=False),
)


BLK = 640


def _tc1_body(x_ref, w_ref, d0_ref, d1_ref, hs_ref, dinv_ref):
    deg = d0_ref[...] + d1_ref[...] + 1.0
    dinv = lax.rsqrt(deg)
    h = jnp.dot(x_ref[...], w_ref[...], preferred_element_type=jnp.float32)
    hs_ref[...] = h * dinv
    dinv_ref[...] = dinv


_tc1 = pl.pallas_call(
    _tc1_body,
    grid=(NPAD // BLK,),
    in_specs=[
        pl.BlockSpec((BLK, D_IN), lambda i: (i, 0)),
        pl.BlockSpec((D_IN, D_HID), lambda i: (0, 0)),
        pl.BlockSpec((BLK, 1), lambda i: (i, 0)),
        pl.BlockSpec((BLK, 1), lambda i: (i, 0)),
    ],
    out_specs=[
        pl.BlockSpec((BLK, D_HID), lambda i: (i, 0)),
        pl.BlockSpec((BLK, 1), lambda i: (i, 0)),
    ],
    out_shape=[
        jax.ShapeDtypeStruct((NPAD, D_HID), jnp.float32),
        jax.ShapeDtypeStruct((NPAD, 1), jnp.float32),
    ],
)


def _tc2_body(a0_ref, a1_ref, hs_ref, dinv_ref, b_ref, w_ref, out_ref):
    dinv = dinv_ref[...]
    pre = (a0_ref[...] + a1_ref[...] + hs_ref[...]) * dinv + b_ref[...]
    t = jnp.maximum(pre, 0.0)
    out_ref[...] = jnp.dot(t, w_ref[...],
                           preferred_element_type=jnp.float32) * dinv


_tc2 = pl.pallas_call(
    _tc2_body,
    grid=(NPAD // BLK,),
    in_specs=[
        pl.BlockSpec((BLK, D_HID), lambda i: (i, 0)),
        pl.BlockSpec((BLK, D_HID), lambda i: (i, 0)),
        pl.BlockSpec((BLK, D_HID), lambda i: (i, 0)),
        pl.BlockSpec((BLK, 1), lambda i: (i, 0)),
        pl.BlockSpec((1, D_HID), lambda i: (0, 0)),
        pl.BlockSpec((D_HID, D_HID), lambda i: (0, 0)),
    ],
    out_specs=pl.BlockSpec((BLK, D_HID), lambda i: (i, 0)),
    out_shape=jax.ShapeDtypeStruct((NPAD, D_HID), jnp.float32),
)


def _tc3_body(a0_ref, a1_ref, hs_ref, dinv_ref, b_ref, w_ref, bf_ref,
              out_ref):
    dinv = dinv_ref[...]
    pre = (a0_ref[...] + a1_ref[...] + hs_ref[...]) * dinv + b_ref[...]
    t = jnp.maximum(pre, 0.0)
    out_ref[...] = jnp.dot(t, w_ref[...],
                           preferred_element_type=jnp.float32) + bf_ref[...]


_tc3 = pl.pallas_call(
    _tc3_body,
    grid=(NPAD // BLK,),
    in_specs=[
        pl.BlockSpec((BLK, D_HID), lambda i: (i, 0)),
        pl.BlockSpec((BLK, D_HID), lambda i: (i, 0)),
        pl.BlockSpec((BLK, D_HID), lambda i: (i, 0)),
        pl.BlockSpec((BLK, 1), lambda i: (i, 0)),
        pl.BlockSpec((1, D_HID), lambda i: (0, 0)),
        pl.BlockSpec((D_HID, OUTP), lambda i: (0, 0)),
        pl.BlockSpec((1, OUTP), lambda i: (0, 0)),
    ],
    out_specs=pl.BlockSpec((BLK, OUTP), lambda i: (i, 0)),
    out_shape=jax.ShapeDtypeStruct((NPAD, OUTP), jnp.float32),
)


@jax.jit
def kernel(x, edge_index, W1, b1, W2, b2, Wf, bf):
    src = edge_index[0].astype(jnp.int32).reshape(NW, NCHUNK, CHUNK)
    dst = edge_index[1].astype(jnp.int32).reshape(NW, NCHUNK, CHUNK)
    dst_flat = dst.reshape(NW, EPW)
    zpad = jnp.zeros((NPAD,), jnp.float32)
    znd = jnp.zeros((NPAD, D_HID), jnp.float32)
    xp = jnp.pad(x, ((0, NPAD - N), (0, 0)))

    deg_parts = _deg(dst_flat, zpad)
    d0 = deg_parts[0].reshape(NPAD, 1)
    d1 = deg_parts[1].reshape(NPAD, 1)

    hs1, dinv = _tc1(xp, W1, d0, d1)
    acc1 = _mp(hs1, src, dst, znd)
    hs2 = _tc2(acc1[0], acc1[1], hs1, dinv, b1.reshape(1, D_HID), W2)
    acc2 = _mp(hs2, src, dst, znd)

    Wfp = jnp.zeros((D_HID, OUTP), jnp.float32).at[:, :D_OUT].set(Wf)
    bfp = jnp.zeros((1, OUTP), jnp.float32).at[:, :D_OUT].set(bf)
    outp = _tc3(acc2[0], acc2[1], hs2, dinv, b2.reshape(1, D_HID), Wfp, bfp)
    return outp[:N, :D_OUT]
